# baseline (speedup 1.0000x reference)
import functools

import jax
import jax.numpy as jnp
from jax.experimental import pallas as pl
from jax.experimental.pallas import tpu as pltpu


def _pass1_body(a_ref, x_ref, w1_ref, b1_ref, w2_ref, adj_ref, m_ref, c_ref):
    @pl.when(pl.program_id(0) == 0)
    def _():
        c_ref[...] = jnp.dot(x_ref[...], w1_ref[...],
                             preferred_element_type=jnp.float32)

    h = jnp.dot(adj_ref[...], c_ref[...],
                preferred_element_type=jnp.float32) + b1_ref[...]
    h = jnp.where(h >= 0, h, a_ref[0, 0] * h)
    m_ref[...] = jnp.dot(h, w2_ref[...], preferred_element_type=jnp.float32)


def _pass2_body(adj_ref, m_ref, b2_ref, z_ref):
    z_ref[...] = jnp.dot(adj_ref[...], m_ref[...],
                         preferred_element_type=jnp.float32) + b2_ref[...]


def _pass3_body(z_ref, zt_ref, out_ref):
    out_ref[...] = jnp.dot(z_ref[...], zt_ref[...],
                           preferred_element_type=jnp.float32)


@jax.jit
def kernel(x, adj, W1, b1, W2, b2, prelu_a):
    N, D = x.shape
    H = W1.shape[1]
    L = W2.shape[1]
    BM = 400
    grid = (N // BM,)

    a2 = prelu_a.reshape(1, 1)
    b1r = b1.reshape(1, H)
    b2r = b2.reshape(1, L)

    m = pl.pallas_call(
        _pass1_body,
        grid=grid,
        in_specs=[
            pl.BlockSpec(memory_space=pltpu.SMEM),
            pl.BlockSpec((N, D), lambda i: (0, 0)),
            pl.BlockSpec((D, H), lambda i: (0, 0)),
            pl.BlockSpec((1, H), lambda i: (0, 0)),
            pl.BlockSpec((H, L), lambda i: (0, 0)),
            pl.BlockSpec((BM, N), lambda i: (i, 0)),
        ],
        out_specs=pl.BlockSpec((BM, L), lambda i: (i, 0)),
        out_shape=jax.ShapeDtypeStruct((N, L), jnp.float32),
        scratch_shapes=[pltpu.VMEM((N, H), jnp.float32)],
    )(a2, x, W1, b1r, W2, adj)

    z = pl.pallas_call(
        _pass2_body,
        grid=grid,
        in_specs=[
            pl.BlockSpec((BM, N), lambda i: (i, 0)),
            pl.BlockSpec((N, L), lambda i: (0, 0)),
            pl.BlockSpec((1, L), lambda i: (0, 0)),
        ],
        out_specs=pl.BlockSpec((BM, L), lambda i: (i, 0)),
        out_shape=jax.ShapeDtypeStruct((N, L), jnp.float32),
    )(adj, m, b2r)

    zt = z.T

    adj_hat = pl.pallas_call(
        _pass3_body,
        grid=grid,
        in_specs=[
            pl.BlockSpec((BM, L), lambda i: (i, 0)),
            pl.BlockSpec((L, N), lambda i: (0, 0)),
        ],
        out_specs=pl.BlockSpec((BM, N), lambda i: (i, 0)),
        out_shape=jax.ShapeDtypeStruct((N, N), jnp.float32),
    )(z, zt)

    return adj_hat
